# Initial kernel scaffold; baseline (speedup 1.0000x reference)
import jax
import jax.numpy as jnp
from jax.experimental import pallas as pl


def kernel(x, edge_index, W1, b1, W2, b2):
    raise NotImplementedError("write your pallas kernel here")


# speedup vs baseline: 7.2702x; 7.2702x over previous
import functools

import jax
import jax.numpy as jnp
from jax import lax
from jax.experimental import pallas as pl
from jax.experimental.pallas import tpu as pltpu
from jax.experimental.pallas import tpu_sc as plsc

NC = 2
NS = 16
NW = NC * NS
LANES = 16
CHUNK = 128


def _sc_mesh():
    return plsc.VectorSubcoreMesh(core_axis_name="c", subcore_axis_name="s")


def _fill_rows(buf, nrows, width, value):
    val = jnp.full((LANES,), value, jnp.float32)

    def body(i, _):
        for k in range(width // LANES):
            buf[i, pl.ds(k * LANES, LANES)] = val
        return 0

    lax.fori_loop(0, nrows, body, 0)


def _make_deg_kernel(rows, cpt):
    @functools.partial(
        pl.kernel,
        mesh=_sc_mesh(),
        out_type=jax.ShapeDtypeStruct((NC, rows, 128), jnp.float32),
        scratch_types=[
            pltpu.VMEM((cpt, CHUNK), jnp.int32),
            pltpu.VMEM((CHUNK, 128), jnp.float32),
            pltpu.VMEM_SHARED((rows, 128), jnp.float32),
        ],
    )
    def deg_kernel(dst_hbm, out_hbm, idx_v, rows_v, deg_sh):
        c = lax.axis_index("c")
        s = lax.axis_index("s")
        wid = c * NS + s

        _fill_rows(rows_v, CHUNK, 128, 0.0)
        nblk = rows // 128
        kmax = -(-nblk // NS)

        def zblk(k, _):
            b = s + k * NS

            @pl.when(b < nblk)
            def _():
                pltpu.sync_copy(rows_v, deg_sh.at[pl.ds(b * 128, 128)])

            return 0

        lax.fori_loop(0, kmax, zblk, 0)
        plsc.subcore_barrier()

        _fill_rows(rows_v, CHUNK, 128, 1.0)
        pltpu.sync_copy(dst_hbm.at[pl.ds(wid * cpt, cpt)], idx_v)

        def body(j, _):
            pltpu.sync_copy(rows_v, deg_sh.at[idx_v.at[j]], add=True)
            return 0

        lax.fori_loop(0, cpt, body, 0)
        plsc.subcore_barrier()

        def oblk(k, _):
            b = s + k * NS

            @pl.when(b < nblk)
            def _():
                pltpu.sync_copy(deg_sh.at[pl.ds(b * 128, 128)], rows_v)
                pltpu.sync_copy(rows_v, out_hbm.at[c, pl.ds(b * 128, 128)])

            return 0

        lax.fori_loop(0, kmax, oblk, 0)

    return deg_kernel


def _make_agg_kernel(rows, cpt):
    @functools.partial(
        pl.kernel,
        mesh=_sc_mesh(),
        out_type=jax.ShapeDtypeStruct((NC, rows, 128), jnp.float32),
        scratch_types=[
            pltpu.VMEM((cpt, CHUNK), jnp.int32),
            pltpu.VMEM((cpt, CHUNK), jnp.int32),
            pltpu.VMEM((CHUNK, 128), jnp.float32),
            pltpu.VMEM_SHARED((rows, 128), jnp.float32),
            pltpu.SemaphoreType.DMA,
        ],
    )
    def agg_kernel(h_hbm, src_hbm, dst_hbm, out_hbm,
                   src_v, dst_v, rows_v, agg_sh, sem):
        c = lax.axis_index("c")
        s = lax.axis_index("s")
        wid = c * NS + s

        _fill_rows(rows_v, CHUNK, 128, 0.0)
        nblk = rows // 128
        kmax = -(-nblk // NS)

        def zblk(k, _):
            b = s + k * NS

            @pl.when(b < nblk)
            def _():
                pltpu.sync_copy(rows_v, agg_sh.at[pl.ds(b * 128, 128)])

            return 0

        lax.fori_loop(0, kmax, zblk, 0)
        plsc.subcore_barrier()

        pltpu.sync_copy(src_hbm.at[pl.ds(wid * cpt, cpt)], src_v)
        pltpu.sync_copy(dst_hbm.at[pl.ds(wid * cpt, cpt)], dst_v)

        def body(j, _):
            pltpu.async_copy(h_hbm.at[src_v.at[j]], rows_v, sem).wait()
            pltpu.sync_copy(rows_v, agg_sh.at[dst_v.at[j]], add=True)
            return 0

        lax.fori_loop(0, cpt, body, 0)
        plsc.subcore_barrier()

        def oblk(k, _):
            b = s + k * NS

            @pl.when(b < nblk)
            def _():
                pltpu.sync_copy(agg_sh.at[pl.ds(b * 128, 128)], rows_v)
                pltpu.sync_copy(rows_v, out_hbm.at[c, pl.ds(b * 128, 128)])

            return 0

        lax.fori_loop(0, kmax, oblk, 0)

    return agg_kernel


def _tc_scale1(x_pad, W1, degp, n):
    rows = x_pad.shape[0]
    grid = rows // 128

    def body(x_ref, w_ref, deg_ref, dinv_ref, hp_ref):
        i = pl.program_id(0)
        deg = deg_ref[0, :, :1] + deg_ref[1, :, :1] + 1.0
        row_id = lax.broadcasted_iota(jnp.int32, (128, LANES), 0) + i * 128
        dinv = jnp.where(row_id < n, lax.rsqrt(deg), 0.0)
        dinv_ref[...] = dinv
        h = jnp.dot(x_ref[...], w_ref[...], preferred_element_type=jnp.float32)
        hp_ref[...] = h * dinv[:, :1]

    return pl.pallas_call(
        body,
        grid=(grid,),
        in_specs=[
            pl.BlockSpec((128, 128), lambda i: (i, 0)),
            pl.BlockSpec((128, 128), lambda i: (0, 0)),
            pl.BlockSpec((NC, 128, 128), lambda i: (0, i, 0)),
        ],
        out_specs=[
            pl.BlockSpec((128, LANES), lambda i: (i, 0)),
            pl.BlockSpec((128, 128), lambda i: (i, 0)),
        ],
        out_shape=[
            jax.ShapeDtypeStruct((rows, LANES), jnp.float32),
            jax.ShapeDtypeStruct((rows, 128), jnp.float32),
        ],
    )(x_pad, W1, degp)


def _tc_mid(aggp, hp, dinv16, W2, b1):
    rows = hp.shape[0]
    grid = rows // 128

    def body(agg_ref, hp_ref, dinv_ref, w_ref, b_ref, out_ref):
        dinv = dinv_ref[...][:, :1]
        pre = (agg_ref[0] + agg_ref[1] + hp_ref[...]) * dinv + b_ref[...]
        h1 = jnp.maximum(pre, 0.0)
        out_ref[...] = jnp.dot(
            h1, w_ref[...], preferred_element_type=jnp.float32) * dinv

    return pl.pallas_call(
        body,
        grid=(grid,),
        in_specs=[
            pl.BlockSpec((NC, 128, 128), lambda i: (0, i, 0)),
            pl.BlockSpec((128, 128), lambda i: (i, 0)),
            pl.BlockSpec((128, LANES), lambda i: (i, 0)),
            pl.BlockSpec((128, 128), lambda i: (0, 0)),
            pl.BlockSpec((1, 128), lambda i: (0, 0)),
        ],
        out_specs=pl.BlockSpec((128, 128), lambda i: (i, 0)),
        out_shape=jax.ShapeDtypeStruct((rows, 128), jnp.float32),
    )(aggp, hp, dinv16, W2, b1)


def _tc_final(aggp, hp, dinv16, b2):
    rows = hp.shape[0]
    grid = rows // 128

    def body(agg_ref, hp_ref, dinv_ref, b_ref, out_ref):
        dinv = dinv_ref[...][:, :1]
        out_ref[...] = (agg_ref[0] + agg_ref[1] + hp_ref[...]) * dinv + b_ref[...]

    return pl.pallas_call(
        body,
        grid=(grid,),
        in_specs=[
            pl.BlockSpec((NC, 128, 128), lambda i: (0, i, 0)),
            pl.BlockSpec((128, 128), lambda i: (i, 0)),
            pl.BlockSpec((128, LANES), lambda i: (i, 0)),
            pl.BlockSpec((1, 128), lambda i: (0, 0)),
        ],
        out_specs=pl.BlockSpec((128, 128), lambda i: (i, 0)),
        out_shape=jax.ShapeDtypeStruct((rows, 128), jnp.float32),
    )(aggp, hp, dinv16, b2)


def kernel(x, edge_index, W1, b1, W2, b2):
    n, d = x.shape
    e = edge_index.shape[1]

    rows = ((n + 1 + 127) // 128) * 128
    while rows % (NS * 2) != 0:
        rows += 128

    cpt = -(-e // (NW * CHUNK))
    cpt = ((cpt + 7) // 8) * 8
    e_pad = NW * CHUNK * cpt

    src = edge_index[0]
    dst = edge_index[1]
    pad = jnp.full((e_pad - e,), n, dtype=jnp.int32)
    src2 = jnp.concatenate([src, pad]).reshape(e_pad // CHUNK, CHUNK)
    dst2 = jnp.concatenate([dst, pad]).reshape(e_pad // CHUNK, CHUNK)
    x_pad = jnp.pad(x, ((0, rows - n), (0, 0)))
    b1r = b1.reshape(1, d)
    b2r = b2.reshape(1, d)

    deg_kernel = _make_deg_kernel(rows, cpt)
    agg_kernel = _make_agg_kernel(rows, cpt)

    degp = deg_kernel(dst2)
    dinv16, h1p = _tc_scale1(x_pad, W1, degp, n)
    agg1 = agg_kernel(h1p, src2, dst2)
    h2p = _tc_mid(agg1, h1p, dinv16, W2, b1r)
    agg2 = agg_kernel(h2p, src2, dst2)
    out = _tc_final(agg2, h2p, dinv16, b2r)
    return out[:n]


# speedup vs baseline: 7.9282x; 1.0905x over previous
import functools

import jax
import jax.numpy as jnp
from jax import lax
from jax.experimental import pallas as pl
from jax.experimental.pallas import tpu as pltpu
from jax.experimental.pallas import tpu_sc as plsc

NC = 2
NS = 16
NW = NC * NS
LANES = 16
CHUNK = 128


def _sc_mesh():
    return plsc.VectorSubcoreMesh(core_axis_name="c", subcore_axis_name="s")


def _fill_rows(buf, nrows, width, value):
    val = jnp.full((LANES,), value, jnp.float32)

    def body(i, _):
        for k in range(width // LANES):
            buf[i, pl.ds(k * LANES, LANES)] = val
        return 0

    lax.fori_loop(0, nrows, body, 0)


def _make_deg_kernel(rows, cpt):
    @functools.partial(
        pl.kernel,
        mesh=_sc_mesh(),
        out_type=jax.ShapeDtypeStruct((NC, rows, 128), jnp.float32),
        scratch_types=[
            pltpu.VMEM((cpt, CHUNK), jnp.int32),
            pltpu.VMEM((CHUNK, 128), jnp.float32),
            pltpu.VMEM_SHARED((rows, 128), jnp.float32),
            pltpu.SemaphoreType.DMA,
        ],
    )
    def deg_kernel(dst_hbm, out_hbm, idx_v, rows_v, deg_sh, sem):
        c = lax.axis_index("c")
        s = lax.axis_index("s")
        wid = c * NS + s

        _fill_rows(rows_v, CHUNK, 128, 0.0)
        nblk = rows // CHUNK
        kmax = -(-nblk // NS)

        def zblk(k, _):
            b = s + k * NS

            @pl.when(b < nblk)
            def _():
                pltpu.sync_copy(rows_v, deg_sh.at[pl.ds(b * CHUNK, CHUNK)])

            return 0

        lax.fori_loop(0, kmax, zblk, 0)
        plsc.subcore_barrier()

        _fill_rows(rows_v, CHUNK, 128, 1.0)
        pltpu.sync_copy(dst_hbm.at[pl.ds(wid * cpt, cpt)], idx_v)

        def body(g, _):
            for b in range(8):
                pltpu.make_async_copy(
                    rows_v, deg_sh.at[idx_v.at[g * 8 + b]], sem,
                ).start(add=True)
            for b in range(8):
                pltpu.make_async_copy(
                    rows_v, deg_sh.at[idx_v.at[g * 8 + b]], sem,
                ).wait()
            return 0

        lax.fori_loop(0, cpt // 8, body, 0)
        plsc.subcore_barrier()

        def oblk(k, _):
            b = s + k * NS

            @pl.when(b < nblk)
            def _():
                pltpu.sync_copy(deg_sh.at[pl.ds(b * CHUNK, CHUNK)], rows_v)
                pltpu.sync_copy(rows_v, out_hbm.at[c, pl.ds(b * CHUNK, CHUNK)])

            return 0

        lax.fori_loop(0, kmax, oblk, 0)

    return deg_kernel


def _make_agg_kernel(rows, cpt):
    @functools.partial(
        pl.kernel,
        mesh=_sc_mesh(),
        out_type=jax.ShapeDtypeStruct((NC, rows, 128), jnp.float32),
        scratch_types=[
            pltpu.VMEM((cpt // 2, CHUNK), jnp.int32),
            pltpu.VMEM((cpt // 2, CHUNK), jnp.int32),
            pltpu.VMEM((CHUNK, 128), jnp.float32),
            pltpu.VMEM((CHUNK, 128), jnp.float32),
            pltpu.VMEM_SHARED((rows, 128), jnp.float32),
            pltpu.SemaphoreType.DMA,
            pltpu.SemaphoreType.DMA,
        ],
    )
    def agg_kernel(h_hbm, src_hbm, dst_hbm, out_hbm,
                   src_v, dst_v, rows_v, rows_w, agg_sh, sem0, sem1):
        c = lax.axis_index("c")
        s = lax.axis_index("s")
        wid = c * NS + s

        _fill_rows(rows_v, CHUNK, 128, 0.0)
        nblk = rows // CHUNK
        kmax = -(-nblk // NS)

        def zblk(k, _):
            b = s + k * NS

            @pl.when(b < nblk)
            def _():
                pltpu.sync_copy(rows_v, agg_sh.at[pl.ds(b * CHUNK, CHUNK)])

            return 0

        lax.fori_loop(0, kmax, zblk, 0)
        plsc.subcore_barrier()

        bufs = (rows_v, rows_w)
        sems = (sem0, sem1)
        hp = cpt // 2

        for p in range(2):
            base = wid * cpt + p * hp
            pltpu.sync_copy(src_hbm.at[pl.ds(base, hp)], src_v)
            pltpu.sync_copy(dst_hbm.at[pl.ds(base, hp)], dst_v)
            pltpu.make_async_copy(h_hbm.at[src_v.at[0]], rows_v, sem0).start()

            def body(i, _):
                j0 = i * 2
                for b in range(2):
                    j = j0 + b

                    @pl.when(j + 1 < hp)
                    def _():
                        pltpu.make_async_copy(
                            h_hbm.at[src_v.at[j + 1]], bufs[1 - b],
                            sems[1 - b],
                        ).start()

                    pltpu.make_async_copy(
                        h_hbm.at[src_v.at[j]], bufs[b], sems[b]).wait()
                    pltpu.sync_copy(bufs[b], agg_sh.at[dst_v.at[j]], add=True)
                return 0

            lax.fori_loop(0, hp // 2, body, 0)
        plsc.subcore_barrier()

        def oblk(k, _):
            b = s + k * NS

            @pl.when(b < nblk)
            def _():
                pltpu.sync_copy(agg_sh.at[pl.ds(b * CHUNK, CHUNK)], rows_v)
                pltpu.sync_copy(rows_v, out_hbm.at[c, pl.ds(b * CHUNK, CHUNK)])

            return 0

        lax.fori_loop(0, kmax, oblk, 0)

    return agg_kernel


def _tc_scale1(x_pad, W1, degp, n):
    rows = x_pad.shape[0]
    grid = rows // 128

    def body(x_ref, w_ref, deg_ref, dinv_ref, hp_ref):
        i = pl.program_id(0)
        deg = deg_ref[0, :, :1] + deg_ref[1, :, :1] + 1.0
        row_id = lax.broadcasted_iota(jnp.int32, (128, LANES), 0) + i * 128
        dinv = jnp.where(row_id < n, lax.rsqrt(deg), 0.0)
        dinv_ref[...] = dinv
        h = jnp.dot(x_ref[...], w_ref[...], preferred_element_type=jnp.float32)
        hp_ref[...] = h * dinv[:, :1]

    return pl.pallas_call(
        body,
        grid=(grid,),
        in_specs=[
            pl.BlockSpec((128, 128), lambda i: (i, 0)),
            pl.BlockSpec((128, 128), lambda i: (0, 0)),
            pl.BlockSpec((NC, 128, 128), lambda i: (0, i, 0)),
        ],
        out_specs=[
            pl.BlockSpec((128, LANES), lambda i: (i, 0)),
            pl.BlockSpec((128, 128), lambda i: (i, 0)),
        ],
        out_shape=[
            jax.ShapeDtypeStruct((rows, LANES), jnp.float32),
            jax.ShapeDtypeStruct((rows, 128), jnp.float32),
        ],
    )(x_pad, W1, degp)


def _tc_mid(aggp, hp, dinv16, W2, b1):
    rows = hp.shape[0]
    grid = rows // 128

    def body(agg_ref, hp_ref, dinv_ref, w_ref, b_ref, out_ref):
        dinv = dinv_ref[...][:, :1]
        pre = (agg_ref[0] + agg_ref[1] + hp_ref[...]) * dinv + b_ref[...]
        h1 = jnp.maximum(pre, 0.0)
        out_ref[...] = jnp.dot(
            h1, w_ref[...], preferred_element_type=jnp.float32) * dinv

    return pl.pallas_call(
        body,
        grid=(grid,),
        in_specs=[
            pl.BlockSpec((NC, 128, 128), lambda i: (0, i, 0)),
            pl.BlockSpec((128, 128), lambda i: (i, 0)),
            pl.BlockSpec((128, LANES), lambda i: (i, 0)),
            pl.BlockSpec((128, 128), lambda i: (0, 0)),
            pl.BlockSpec((1, 128), lambda i: (0, 0)),
        ],
        out_specs=pl.BlockSpec((128, 128), lambda i: (i, 0)),
        out_shape=jax.ShapeDtypeStruct((rows, 128), jnp.float32),
    )(aggp, hp, dinv16, W2, b1)


def _tc_final(aggp, hp, dinv16, b2):
    rows = hp.shape[0]
    grid = rows // 128

    def body(agg_ref, hp_ref, dinv_ref, b_ref, out_ref):
        dinv = dinv_ref[...][:, :1]
        out_ref[...] = (agg_ref[0] + agg_ref[1] + hp_ref[...]) * dinv + b_ref[...]

    return pl.pallas_call(
        body,
        grid=(grid,),
        in_specs=[
            pl.BlockSpec((NC, 128, 128), lambda i: (0, i, 0)),
            pl.BlockSpec((128, 128), lambda i: (i, 0)),
            pl.BlockSpec((128, LANES), lambda i: (i, 0)),
            pl.BlockSpec((1, 128), lambda i: (0, 0)),
        ],
        out_specs=pl.BlockSpec((128, 128), lambda i: (i, 0)),
        out_shape=jax.ShapeDtypeStruct((rows, 128), jnp.float32),
    )(aggp, hp, dinv16, b2)


def kernel(x, edge_index, W1, b1, W2, b2):
    n, d = x.shape
    e = edge_index.shape[1]

    rows = ((n + 1 + 127) // 128) * 128
    while rows % (NS * 2) != 0:
        rows += 128

    cpt = -(-e // (NW * CHUNK))
    cpt = ((cpt + 7) // 8) * 8
    e_pad = NW * CHUNK * cpt

    src = edge_index[0]
    dst = edge_index[1]
    pad = jnp.full((e_pad - e,), n, dtype=jnp.int32)
    src2 = jnp.concatenate([src, pad]).reshape(e_pad // CHUNK, CHUNK)
    dst2 = jnp.concatenate([dst, pad]).reshape(e_pad // CHUNK, CHUNK)
    x_pad = jnp.pad(x, ((0, rows - n), (0, 0)))
    b1r = b1.reshape(1, d)
    b2r = b2.reshape(1, d)

    deg_kernel = _make_deg_kernel(rows, cpt)
    agg_kernel = _make_agg_kernel(rows, cpt)

    degp = deg_kernel(dst2)
    dinv16, h1p = _tc_scale1(x_pad, W1, degp, n)
    agg1 = agg_kernel(h1p, src2, dst2)
    h2p = _tc_mid(agg1, h1p, dinv16, W2, b1r)
    agg2 = agg_kernel(h2p, src2, dst2)
    out = _tc_final(agg2, h2p, dinv16, b2r)
    return out[:n]


# speedup vs baseline: 8.3289x; 1.0505x over previous
import functools

import jax
import jax.numpy as jnp
from jax import lax
from jax.experimental import pallas as pl
from jax.experimental.pallas import tpu as pltpu
from jax.experimental.pallas import tpu_sc as plsc

NC = 2
NS = 16
NW = NC * NS
LANES = 16
CHUNK = 128


def _sc_mesh():
    return plsc.VectorSubcoreMesh(core_axis_name="c", subcore_axis_name="s")


def _fill_rows(buf, nrows, width, value):
    val = jnp.full((LANES,), value, jnp.float32)

    def body(i, _):
        for k in range(width // LANES):
            buf[i, pl.ds(k * LANES, LANES)] = val
        return 0

    lax.fori_loop(0, nrows, body, 0)


def _make_deg_kernel(rows, cpt):
    @functools.partial(
        pl.kernel,
        mesh=_sc_mesh(),
        out_type=jax.ShapeDtypeStruct((NC, rows, 128), jnp.float32),
        scratch_types=[
            pltpu.VMEM((cpt, CHUNK), jnp.int32),
            pltpu.VMEM((CHUNK, 128), jnp.float32),
            pltpu.VMEM_SHARED((rows, 128), jnp.float32),
            pltpu.SemaphoreType.DMA,
        ],
    )
    def deg_kernel(dst_hbm, out_hbm, idx_v, rows_v, deg_sh, sem):
        c = lax.axis_index("c")
        s = lax.axis_index("s")
        wid = c * NS + s

        _fill_rows(rows_v, CHUNK, 128, 0.0)
        nblk = rows // CHUNK
        kmax = -(-nblk // NS)

        def zblk(k, _):
            b = s + k * NS

            @pl.when(b < nblk)
            def _():
                pltpu.sync_copy(rows_v, deg_sh.at[pl.ds(b * CHUNK, CHUNK)])

            return 0

        lax.fori_loop(0, kmax, zblk, 0)
        plsc.subcore_barrier()

        _fill_rows(rows_v, CHUNK, 128, 1.0)
        pltpu.sync_copy(dst_hbm.at[pl.ds(wid * cpt, cpt)], idx_v)

        def body(g, _):
            for b in range(8):
                pltpu.make_async_copy(
                    rows_v, deg_sh.at[idx_v.at[g * 8 + b]], sem,
                ).start(add=True)
            for b in range(8):
                pltpu.make_async_copy(
                    rows_v, deg_sh.at[idx_v.at[g * 8 + b]], sem,
                ).wait()
            return 0

        lax.fori_loop(0, cpt // 8, body, 0)
        plsc.subcore_barrier()

        def oblk(k, _):
            b = s + k * NS

            @pl.when(b < nblk)
            def _():
                pltpu.sync_copy(deg_sh.at[pl.ds(b * CHUNK, CHUNK)], rows_v)
                pltpu.sync_copy(rows_v, out_hbm.at[c, pl.ds(b * CHUNK, CHUNK)])

            return 0

        lax.fori_loop(0, kmax, oblk, 0)

    return deg_kernel


def _make_agg_kernel(rows, cpt_a, cpt_b):
    cpt_max = max(cpt_a, cpt_b)

    @functools.partial(
        pl.kernel,
        mesh=_sc_mesh(),
        out_type=jax.ShapeDtypeStruct((NC, rows, 128), jnp.float32),
        scratch_types=[
            pltpu.VMEM((cpt_max // 2, CHUNK), jnp.int32),
            pltpu.VMEM((cpt_max // 2, CHUNK), jnp.int32),
            pltpu.VMEM((CHUNK, 128), jnp.float32),
            pltpu.VMEM((CHUNK, 128), jnp.float32),
            pltpu.VMEM_SHARED((rows, 128), jnp.float32),
            pltpu.SemaphoreType.DMA,
            pltpu.SemaphoreType.DMA,
        ],
    )
    def agg_kernel(h_hbm, src_hbm, dst_hbm, out_hbm,
                   src_v, dst_v, rows_v, rows_w, agg_sh, sem0, sem1):
        c = lax.axis_index("c")
        s = lax.axis_index("s")
        my_cpt = jnp.where(c == 0, cpt_a, cpt_b)
        my_base = jnp.where(c == 0, s * cpt_a, NS * cpt_a + s * cpt_b)

        _fill_rows(rows_v, CHUNK, 128, 0.0)
        nblk = rows // CHUNK
        kmax = -(-nblk // NS)

        def zblk(k, _):
            b = s + k * NS

            @pl.when(b < nblk)
            def _():
                pltpu.sync_copy(rows_v, agg_sh.at[pl.ds(b * CHUNK, CHUNK)])

            return 0

        lax.fori_loop(0, kmax, zblk, 0)
        plsc.subcore_barrier()

        bufs = (rows_v, rows_w)
        sems = (sem0, sem1)
        hp = my_cpt // 2

        for p in range(2):
            base = pl.multiple_of(my_base + p * hp, 8)
            pltpu.sync_copy(src_hbm.at[pl.ds(base, cpt_max // 2)], src_v)
            pltpu.sync_copy(dst_hbm.at[pl.ds(base, cpt_max // 2)], dst_v)
            pltpu.make_async_copy(h_hbm.at[src_v.at[0]], rows_v, sem0).start()

            def body(i, _):
                j0 = i * 2
                for b in range(2):
                    j = j0 + b

                    @pl.when(j + 1 < hp)
                    def _():
                        pltpu.make_async_copy(
                            h_hbm.at[src_v.at[j + 1]], bufs[1 - b],
                            sems[1 - b],
                        ).start()

                    pltpu.make_async_copy(
                        h_hbm.at[src_v.at[j]], bufs[b], sems[b]).wait()
                    pltpu.sync_copy(bufs[b], agg_sh.at[dst_v.at[j]], add=True)
                return 0

            lax.fori_loop(0, hp // 2, body, 0)
        plsc.subcore_barrier()

        def oblk(k, _):
            b = s + k * NS

            @pl.when(b < nblk)
            def _():
                pltpu.sync_copy(agg_sh.at[pl.ds(b * CHUNK, CHUNK)], rows_v)
                pltpu.sync_copy(rows_v, out_hbm.at[c, pl.ds(b * CHUNK, CHUNK)])

            return 0

        lax.fori_loop(0, kmax, oblk, 0)

    return agg_kernel


def _tc_scale1(x_pad, W1, degp, n):
    rows = x_pad.shape[0]
    grid = rows // 128

    def body(x_ref, w_ref, deg_ref, dinv_ref, hp_ref):
        i = pl.program_id(0)
        deg = deg_ref[0, :, :1] + deg_ref[1, :, :1] + 1.0
        row_id = lax.broadcasted_iota(jnp.int32, (128, LANES), 0) + i * 128
        dinv = jnp.where(row_id < n, lax.rsqrt(deg), 0.0)
        dinv_ref[...] = dinv
        h = jnp.dot(x_ref[...], w_ref[...], preferred_element_type=jnp.float32)
        hp_ref[...] = h * dinv[:, :1]

    return pl.pallas_call(
        body,
        grid=(grid,),
        in_specs=[
            pl.BlockSpec((128, 128), lambda i: (i, 0)),
            pl.BlockSpec((128, 128), lambda i: (0, 0)),
            pl.BlockSpec((NC, 128, 128), lambda i: (0, i, 0)),
        ],
        out_specs=[
            pl.BlockSpec((128, LANES), lambda i: (i, 0)),
            pl.BlockSpec((128, 128), lambda i: (i, 0)),
        ],
        out_shape=[
            jax.ShapeDtypeStruct((rows, LANES), jnp.float32),
            jax.ShapeDtypeStruct((rows, 128), jnp.float32),
        ],
    )(x_pad, W1, degp)


def _tc_mid(aggp, hp, dinv16, W2, b1):
    rows = hp.shape[0]
    grid = rows // 128

    def body(agg_ref, hp_ref, dinv_ref, w_ref, b_ref, out_ref):
        dinv = dinv_ref[...][:, :1]
        pre = (agg_ref[0] + agg_ref[1] + hp_ref[...]) * dinv + b_ref[...]
        h1 = jnp.maximum(pre, 0.0)
        out_ref[...] = jnp.dot(
            h1, w_ref[...], preferred_element_type=jnp.float32) * dinv

    return pl.pallas_call(
        body,
        grid=(grid,),
        in_specs=[
            pl.BlockSpec((NC, 128, 128), lambda i: (0, i, 0)),
            pl.BlockSpec((128, 128), lambda i: (i, 0)),
            pl.BlockSpec((128, LANES), lambda i: (i, 0)),
            pl.BlockSpec((128, 128), lambda i: (0, 0)),
            pl.BlockSpec((1, 128), lambda i: (0, 0)),
        ],
        out_specs=pl.BlockSpec((128, 128), lambda i: (i, 0)),
        out_shape=jax.ShapeDtypeStruct((rows, 128), jnp.float32),
    )(aggp, hp, dinv16, W2, b1)


def _tc_final(aggp, hp, dinv16, b2):
    rows = hp.shape[0]
    grid = rows // 128

    def body(agg_ref, hp_ref, dinv_ref, b_ref, out_ref):
        dinv = dinv_ref[...][:, :1]
        out_ref[...] = (agg_ref[0] + agg_ref[1] + hp_ref[...]) * dinv + b_ref[...]

    return pl.pallas_call(
        body,
        grid=(grid,),
        in_specs=[
            pl.BlockSpec((NC, 128, 128), lambda i: (0, i, 0)),
            pl.BlockSpec((128, 128), lambda i: (i, 0)),
            pl.BlockSpec((128, LANES), lambda i: (i, 0)),
            pl.BlockSpec((1, 128), lambda i: (0, 0)),
        ],
        out_specs=pl.BlockSpec((128, 128), lambda i: (i, 0)),
        out_shape=jax.ShapeDtypeStruct((rows, 128), jnp.float32),
    )(aggp, hp, dinv16, b2)


def kernel(x, edge_index, W1, b1, W2, b2):
    n, d = x.shape
    e = edge_index.shape[1]

    rows = ((n + 1 + 127) // 128) * 128
    while rows % (NS * 2) != 0:
        rows += 128

    cpt = -(-e // (NW * CHUNK))
    cpt = ((cpt + 7) // 8) * 8
    e_pad = NW * CHUNK * cpt

    cpt_a = ((2 * cpt * 4 // 5) // 16) * 16
    cpt_b = 2 * cpt - cpt_a
    over = max(cpt_a, cpt_b) // 2

    src = edge_index[0]
    dst = edge_index[1]
    e_arr = e_pad + over * CHUNK
    pad = jnp.full((e_arr - e,), n, dtype=jnp.int32)
    src2 = jnp.concatenate([src, pad]).reshape(e_arr // CHUNK, CHUNK)
    dst2 = jnp.concatenate([dst, pad]).reshape(e_arr // CHUNK, CHUNK)
    x_pad = jnp.pad(x, ((0, rows - n), (0, 0)))
    b1r = b1.reshape(1, d)
    b2r = b2.reshape(1, d)

    deg_kernel = _make_deg_kernel(rows, cpt)
    agg_kernel = _make_agg_kernel(rows, cpt_a, cpt_b)

    degp = deg_kernel(dst2)
    dinv16, h1p = _tc_scale1(x_pad, W1, degp, n)
    agg1 = agg_kernel(h1p, src2, dst2)
    h2p = _tc_mid(agg1, h1p, dinv16, W2, b1r)
    agg2 = agg_kernel(h2p, src2, dst2)
    out = _tc_final(agg2, h2p, dinv16, b2r)
    return out[:n]
